# baseline (speedup 1.0000x reference)
import functools

import jax
import jax.numpy as jnp
from jax import lax
from jax.experimental import pallas as pl
from jax.experimental.pallas import tpu as pltpu
from jax.experimental.pallas import tpu_sc as plsc

N = 10000
NPAD = 10240
E = 320000
D = 128
DOUT = 64

NC = 2
NS = 16
NW = NC * NS
B = 128
NB = 80
EPAD = NW * NB * B
EPT = NB * B
RPS = NPAD // NS
DEGW = 128

_MESH = plsc.VectorSubcoreMesh(core_axis_name="c", subcore_axis_name="s",
                               num_cores=NC, num_subcores=NS)


def _make_edge_scatter(width):
  @functools.partial(
      pl.kernel,
      out_type=jax.ShapeDtypeStruct((NC, NPAD, width), jnp.float32),
      mesh=_MESH,
      scratch_types=[
          pltpu.VMEM((NB, B), jnp.int32),
          pltpu.VMEM((NB, B), jnp.int32),
          pltpu.VMEM((B, width), jnp.float32),
          pltpu.VMEM_SHARED((NPAD, width), jnp.float32),
          pltpu.SemaphoreType.DMA,
      ],
  )
  def scat(g_hbm, src_hbm, dst_hbm, zeros_hbm, out_hbm,
           src_v, dst_v, rows, acc, sem):
    cid = lax.axis_index("c")
    sid = lax.axis_index("s")
    wid = cid * NS + sid
    pltpu.sync_copy(zeros_hbm, acc.at[pl.ds(sid * RPS, RPS)])
    pltpu.sync_copy(src_hbm.at[wid], src_v)
    pltpu.sync_copy(dst_hbm.at[wid], dst_v)
    plsc.subcore_barrier()

    def body(j, carry):
      pltpu.async_copy(g_hbm.at[src_v.at[j]], rows, sem).wait()
      pltpu.sync_copy(rows, acc.at[dst_v.at[j]], add=True)
      return carry

    lax.fori_loop(0, NB, body, 0)
    plsc.subcore_barrier()
    pltpu.sync_copy(acc.at[pl.ds(sid * RPS, RPS)],
                    out_hbm.at[cid, pl.ds(sid * RPS, RPS)])

  return scat


@functools.partial(
    pl.kernel,
    out_type=jax.ShapeDtypeStruct((NC, NPAD, DEGW), jnp.float32),
    mesh=_MESH,
    scratch_types=[
        pltpu.VMEM((NB, B), jnp.int32),
        pltpu.VMEM((B, DEGW), jnp.float32),
        pltpu.VMEM_SHARED((NPAD, DEGW), jnp.float32),
    ],
)
def _degree_kernel(ones_hbm, dst_hbm, zeros_hbm, out_hbm,
                   dst_v, ones_v, acc):
  cid = lax.axis_index("c")
  sid = lax.axis_index("s")
  wid = cid * NS + sid
  pltpu.sync_copy(zeros_hbm, acc.at[pl.ds(sid * RPS, RPS)])
  pltpu.sync_copy(dst_hbm.at[wid], dst_v)
  pltpu.sync_copy(ones_hbm, ones_v)
  plsc.subcore_barrier()

  def body(j, carry):
    pltpu.sync_copy(ones_v, acc.at[dst_v.at[j]], add=True)
    return carry

  lax.fori_loop(0, NB, body, 0)
  plsc.subcore_barrier()
  pltpu.sync_copy(acc.at[pl.ds(sid * RPS, RPS)],
                  out_hbm.at[cid, pl.ds(sid * RPS, RPS)])


GRID = 8
BR = NPAD // GRID

_row = lambda w: pl.BlockSpec((BR, w), lambda i: (i, 0))
_full = lambda r, w: pl.BlockSpec((r, w), lambda i: (0, 0))


def _stage_a(x_ref, p0_ref, p1_ref, w_ref, g_ref, dis_ref):
  dis = lax.rsqrt(1.0 + p0_ref[...] + p1_ref[...])
  g_ref[...] = jnp.dot(x_ref[...], w_ref[...],
                       preferred_element_type=jnp.float32) * dis
  dis_ref[...] = dis


def _stage_mid(s0_ref, s1_ref, g_ref, dis_ref, b_ref, w_ref, out_ref):
  dis = dis_ref[...]
  h = dis * (s0_ref[...] + s1_ref[...] + g_ref[...]) + b_ref[...]
  h = jnp.maximum(h, 0.0)
  out_ref[...] = jnp.dot(h, w_ref[...],
                         preferred_element_type=jnp.float32) * dis


def _stage_out(s0_ref, s1_ref, g_ref, dis_ref, b_ref, out_ref):
  o = dis_ref[...] * (s0_ref[...] + s1_ref[...] + g_ref[...]) + b_ref[...]
  m = jnp.max(o, axis=1, keepdims=True)
  e = o - m
  out_ref[...] = e - jnp.log(jnp.sum(jnp.exp(e), axis=1, keepdims=True))


def _tc_a(x, p0, p1, w):
  return pl.pallas_call(
      _stage_a,
      grid=(GRID,),
      in_specs=[_row(D), _row(1), _row(1), _full(D, D)],
      out_specs=[_row(D), _row(1)],
      out_shape=[jax.ShapeDtypeStruct((NPAD, D), jnp.float32),
                 jax.ShapeDtypeStruct((NPAD, 1), jnp.float32)],
  )(x, p0, p1, w)


def _tc_mid(s0, s1, g, dis, b, w, dout):
  return pl.pallas_call(
      _stage_mid,
      grid=(GRID,),
      in_specs=[_row(D), _row(D), _row(D), _row(1), _full(1, D), _full(D, dout)],
      out_specs=_row(dout),
      out_shape=jax.ShapeDtypeStruct((NPAD, dout), jnp.float32),
  )(s0, s1, g, dis, b, w)


def _tc_out(s0, s1, g, dis, b):
  return pl.pallas_call(
      _stage_out,
      grid=(GRID,),
      in_specs=[_row(DOUT), _row(DOUT), _row(DOUT), _row(1), _full(1, DOUT)],
      out_specs=_row(DOUT),
      out_shape=jax.ShapeDtypeStruct((NPAD, DOUT), jnp.float32),
  )(s0, s1, g, dis, b)


_scatter_d = _make_edge_scatter(D)


def kernel(x, edge_index, W0, b0, W1, b1, W2, b2, W3, b3):
  pad = EPAD - E
  src = jnp.pad(edge_index[0], (0, pad)).reshape(NW, NB, B)
  dst = jnp.pad(edge_index[1], (0, pad),
                constant_values=N).reshape(NW, NB, B)

  x_pad = jnp.pad(x, ((0, NPAD - N), (0, 0)))
  zeros_d = jnp.zeros((RPS, D), jnp.float32)
  zeros_degw = jnp.zeros((RPS, DEGW), jnp.float32)
  ones_deg = jnp.ones((B, DEGW), jnp.float32)

  deg = _degree_kernel(ones_deg, dst, zeros_degw)
  p0 = deg[0, :, 0:1]
  p1 = deg[1, :, 0:1]

  g0, dis = _tc_a(x_pad, p0, p1, W0)

  s = _scatter_d(g0, src, dst, zeros_d)
  g1 = _tc_mid(s[0], s[1], g0, dis, b0.reshape(1, D), W1, D)

  s = _scatter_d(g1, src, dst, zeros_d)
  g2 = _tc_mid(s[0], s[1], g1, dis, b1.reshape(1, D), W2, D)

  s = _scatter_d(g2, src, dst, zeros_d)
  g3 = _tc_mid(s[0], s[1], g2, dis, b2.reshape(1, D), W3, DOUT)

  g3p = jnp.pad(g3, ((0, 0), (0, D - DOUT)))
  s = _scatter_d(g3p, src, dst, zeros_d)
  out = _tc_out(s[0, :, :DOUT], s[1, :, :DOUT], g3, dis, b3.reshape(1, DOUT))

  return out[:N]


# speedup vs baseline: 1.0005x; 1.0005x over previous
import functools

import jax
import jax.numpy as jnp
from jax import lax
from jax.experimental import pallas as pl
from jax.experimental.pallas import tpu as pltpu
from jax.experimental.pallas import tpu_sc as plsc

N = 10000
NPAD = 10240
E = 320000
D = 128
DOUT = 64

NC = 2
NS = 16
NW = NC * NS
B = 128
NB = 80
EPAD = NW * NB * B
EPT = NB * B
RPS = NPAD // NS
DEGW = 128

_MESH = plsc.VectorSubcoreMesh(core_axis_name="c", subcore_axis_name="s",
                               num_cores=NC, num_subcores=NS)


def _make_edge_scatter(width):
  @functools.partial(
      pl.kernel,
      out_type=jax.ShapeDtypeStruct((NC, NPAD, width), jnp.float32),
      mesh=_MESH,
      scratch_types=[
          pltpu.VMEM((NB, B), jnp.int32),
          pltpu.VMEM((NB, B), jnp.int32),
          pltpu.VMEM((B, width), jnp.float32),
          pltpu.VMEM_SHARED((NPAD, width), jnp.float32),
          pltpu.SemaphoreType.DMA,
      ],
  )
  def scat(g_hbm, src_hbm, dst_hbm, zeros_hbm, out_hbm,
           src_v, dst_v, rows, acc, sem):
    cid = lax.axis_index("c")
    sid = lax.axis_index("s")
    wid = cid * NS + sid
    pltpu.sync_copy(zeros_hbm, acc.at[pl.ds(sid * RPS, RPS)])
    pltpu.sync_copy(src_hbm.at[wid], src_v)
    pltpu.sync_copy(dst_hbm.at[wid], dst_v)
    plsc.subcore_barrier()

    def body(j, carry):
      pltpu.async_copy(g_hbm.at[src_v.at[j]], rows, sem).wait()
      pltpu.sync_copy(rows, acc.at[dst_v.at[j]], add=True)
      return carry

    lax.fori_loop(0, NB, body, 0)
    plsc.subcore_barrier()
    pltpu.sync_copy(acc.at[pl.ds(sid * RPS, RPS)],
                    out_hbm.at[cid, pl.ds(sid * RPS, RPS)])

  return scat


@functools.partial(
    pl.kernel,
    out_type=jax.ShapeDtypeStruct((NC, NPAD, DEGW), jnp.float32),
    mesh=_MESH,
    scratch_types=[
        pltpu.VMEM((NB, B), jnp.int32),
        pltpu.VMEM((B, DEGW), jnp.float32),
        pltpu.VMEM_SHARED((NPAD, DEGW), jnp.float32),
    ],
)
def _degree_kernel(ones_hbm, dst_hbm, zeros_hbm, out_hbm,
                   dst_v, ones_v, acc):
  cid = lax.axis_index("c")
  sid = lax.axis_index("s")
  wid = cid * NS + sid
  pltpu.sync_copy(zeros_hbm, acc.at[pl.ds(sid * RPS, RPS)])
  pltpu.sync_copy(dst_hbm.at[wid], dst_v)
  pltpu.sync_copy(ones_hbm, ones_v)
  plsc.subcore_barrier()

  def body(j, carry):
    pltpu.sync_copy(ones_v, acc.at[dst_v.at[j]], add=True)
    return carry

  lax.fori_loop(0, NB, body, 0)
  plsc.subcore_barrier()
  pltpu.sync_copy(acc.at[pl.ds(sid * RPS, RPS)],
                  out_hbm.at[cid, pl.ds(sid * RPS, RPS)])


GRID = 8
BR = NPAD // GRID

_row = lambda w: pl.BlockSpec((BR, w), lambda i: (i, 0))
_full = lambda r, w: pl.BlockSpec((r, w), lambda i: (0, 0))


def _stage_a(x_ref, p0_ref, p1_ref, w_ref, g_ref, dis_ref):
  dis = lax.rsqrt(1.0 + p0_ref[...] + p1_ref[...])
  g_ref[...] = jnp.dot(x_ref[...], w_ref[...],
                       preferred_element_type=jnp.float32) * dis
  dis_ref[...] = dis


def _stage_mid(s0_ref, s1_ref, g_ref, dis_ref, b_ref, w_ref, out_ref):
  dis = dis_ref[...]
  h = dis * (s0_ref[...] + s1_ref[...] + g_ref[...]) + b_ref[...]
  h = jnp.maximum(h, 0.0)
  out_ref[...] = jnp.dot(h, w_ref[...],
                         preferred_element_type=jnp.float32) * dis


def _stage_out(s0_ref, s1_ref, g_ref, dis_ref, b_ref, out_ref):
  o = dis_ref[...] * (s0_ref[...] + s1_ref[...] + g_ref[...]) + b_ref[...]
  m = jnp.max(o, axis=1, keepdims=True)
  e = o - m
  out_ref[...] = e - jnp.log(jnp.sum(jnp.exp(e), axis=1, keepdims=True))


def _tc_a(x, p0, p1, w):
  return pl.pallas_call(
      _stage_a,
      grid=(GRID,),
      in_specs=[_row(D), _row(1), _row(1), _full(D, D)],
      out_specs=[_row(D), _row(1)],
      out_shape=[jax.ShapeDtypeStruct((NPAD, D), jnp.float32),
                 jax.ShapeDtypeStruct((NPAD, 1), jnp.float32)],
  )(x, p0, p1, w)


def _tc_mid(s0, s1, g, dis, b, w, dout):
  return pl.pallas_call(
      _stage_mid,
      grid=(GRID,),
      in_specs=[_row(D), _row(D), _row(D), _row(1), _full(1, D), _full(D, dout)],
      out_specs=_row(dout),
      out_shape=jax.ShapeDtypeStruct((NPAD, dout), jnp.float32),
  )(s0, s1, g, dis, b, w)


def _tc_out(s0, s1, g, dis, b):
  return pl.pallas_call(
      _stage_out,
      grid=(GRID,),
      in_specs=[_row(DOUT), _row(DOUT), _row(DOUT), _row(1), _full(1, DOUT)],
      out_specs=_row(DOUT),
      out_shape=jax.ShapeDtypeStruct((NPAD, DOUT), jnp.float32),
  )(s0, s1, g, dis, b)


_scatter_d = _make_edge_scatter(D)


def kernel(x, edge_index, W0, b0, W1, b1, W2, b2, W3, b3):
  pad = EPAD - E
  src = jnp.pad(edge_index[0], (0, pad)).reshape(NW, NB, B)
  trash = N + (jnp.arange(pad, dtype=jnp.int32) % (NPAD - N))
  dst = jnp.concatenate([edge_index[1], trash]).reshape(NW, NB, B)

  x_pad = jnp.pad(x, ((0, NPAD - N), (0, 0)))
  zeros_d = jnp.zeros((RPS, D), jnp.float32)
  zeros_degw = jnp.zeros((RPS, DEGW), jnp.float32)
  ones_deg = jnp.ones((B, DEGW), jnp.float32)

  deg = _degree_kernel(ones_deg, dst, zeros_degw)
  p0 = deg[0, :, 0:1]
  p1 = deg[1, :, 0:1]

  g0, dis = _tc_a(x_pad, p0, p1, W0)

  s = _scatter_d(g0, src, dst, zeros_d)
  g1 = _tc_mid(s[0], s[1], g0, dis, b0.reshape(1, D), W1, D)

  s = _scatter_d(g1, src, dst, zeros_d)
  g2 = _tc_mid(s[0], s[1], g1, dis, b1.reshape(1, D), W2, D)

  s = _scatter_d(g2, src, dst, zeros_d)
  g3 = _tc_mid(s[0], s[1], g2, dis, b2.reshape(1, D), W3, DOUT)

  g3p = jnp.pad(g3, ((0, 0), (0, D - DOUT)))
  s = _scatter_d(g3p, src, dst, zeros_d)
  out = _tc_out(s[0, :, :DOUT], s[1, :, :DOUT], g3, dis, b3.reshape(1, DOUT))

  return out[:N]


# speedup vs baseline: 2.4136x; 2.4123x over previous
import functools

import jax
import jax.numpy as jnp
from jax import lax
from jax.experimental import pallas as pl
from jax.experimental.pallas import tpu as pltpu
from jax.experimental.pallas import tpu_sc as plsc

N = 10000
NPAD = 10240
E = 320000
D = 128
DOUT = 64

NC = 2
NS = 16
NW = NC * NS
EPT = E // NW
B = 80
NB = EPT // B
RPS = NPAD // NS
DEGW = 128

_MESH = plsc.VectorSubcoreMesh(core_axis_name="c", subcore_axis_name="s",
                               num_cores=NC, num_subcores=NS)


def _make_edge_scatter(width):
  @functools.partial(
      pl.kernel,
      out_type=jax.ShapeDtypeStruct((NC, NPAD, width), jnp.float32),
      mesh=_MESH,
      scratch_types=[
          pltpu.VMEM((NB, B), jnp.int32),
          pltpu.VMEM((NB, B), jnp.int32),
          pltpu.VMEM((B, width), jnp.float32),
          pltpu.VMEM_SHARED((NPAD, width), jnp.float32),
          pltpu.SemaphoreType.DMA,
      ],
  )
  def scat(g_hbm, src_hbm, dst_hbm, zeros_hbm, out_hbm,
           src_v, dst_v, rows, acc, sem):
    cid = lax.axis_index("c")
    sid = lax.axis_index("s")
    wid = cid * NS + sid
    pltpu.sync_copy(zeros_hbm, acc.at[pl.ds(sid * RPS, RPS)])
    pltpu.sync_copy(src_hbm.at[wid], src_v)
    pltpu.sync_copy(dst_hbm.at[wid], dst_v)
    plsc.subcore_barrier()

    @plsc.parallel_loop(0, NB, unroll=4)
    def body(j):
      pltpu.async_copy(g_hbm.at[src_v.at[j]], rows, sem).wait()
      pltpu.sync_copy(rows, acc.at[dst_v.at[j]], add=True)
    plsc.subcore_barrier()
    pltpu.sync_copy(acc.at[pl.ds(sid * RPS, RPS)],
                    out_hbm.at[cid, pl.ds(sid * RPS, RPS)])

  return scat


@functools.partial(
    pl.kernel,
    out_type=jax.ShapeDtypeStruct((NC, NPAD, DEGW), jnp.float32),
    mesh=_MESH,
    scratch_types=[
        pltpu.VMEM((NB, B), jnp.int32),
        pltpu.VMEM((B, DEGW), jnp.float32),
        pltpu.VMEM_SHARED((NPAD, DEGW), jnp.float32),
    ],
)
def _degree_kernel(ones_hbm, dst_hbm, zeros_hbm, out_hbm,
                   dst_v, ones_v, acc):
  cid = lax.axis_index("c")
  sid = lax.axis_index("s")
  wid = cid * NS + sid
  pltpu.sync_copy(zeros_hbm, acc.at[pl.ds(sid * RPS, RPS)])
  pltpu.sync_copy(dst_hbm.at[wid], dst_v)
  pltpu.sync_copy(ones_hbm, ones_v)
  plsc.subcore_barrier()

  def body(j, carry):
    pltpu.sync_copy(ones_v, acc.at[dst_v.at[j]], add=True)
    return carry

  lax.fori_loop(0, NB, body, 0)
  plsc.subcore_barrier()
  pltpu.sync_copy(acc.at[pl.ds(sid * RPS, RPS)],
                  out_hbm.at[cid, pl.ds(sid * RPS, RPS)])


GRID = 8
BR = NPAD // GRID

_row = lambda w: pl.BlockSpec((BR, w), lambda i: (i, 0))
_full = lambda r, w: pl.BlockSpec((r, w), lambda i: (0, 0))


def _stage_a(x_ref, p0_ref, p1_ref, w_ref, g_ref, dis_ref):
  dis = lax.rsqrt(1.0 + p0_ref[...] + p1_ref[...])
  g_ref[...] = jnp.dot(x_ref[...], w_ref[...],
                       preferred_element_type=jnp.float32) * dis
  dis_ref[...] = dis


def _stage_mid(s0_ref, s1_ref, g_ref, dis_ref, b_ref, w_ref, out_ref):
  dis = dis_ref[...]
  h = dis * (s0_ref[...] + s1_ref[...] + g_ref[...]) + b_ref[...]
  h = jnp.maximum(h, 0.0)
  out_ref[...] = jnp.dot(h, w_ref[...],
                         preferred_element_type=jnp.float32) * dis


def _stage_out(s0_ref, s1_ref, g_ref, dis_ref, b_ref, out_ref):
  o = dis_ref[...] * (s0_ref[...] + s1_ref[...] + g_ref[...]) + b_ref[...]
  m = jnp.max(o, axis=1, keepdims=True)
  e = o - m
  out_ref[...] = e - jnp.log(jnp.sum(jnp.exp(e), axis=1, keepdims=True))


def _tc_a(x, p0, p1, w):
  return pl.pallas_call(
      _stage_a,
      grid=(GRID,),
      in_specs=[_row(D), _row(1), _row(1), _full(D, D)],
      out_specs=[_row(D), _row(1)],
      out_shape=[jax.ShapeDtypeStruct((NPAD, D), jnp.float32),
                 jax.ShapeDtypeStruct((NPAD, 1), jnp.float32)],
  )(x, p0, p1, w)


def _tc_mid(s0, s1, g, dis, b, w, dout):
  return pl.pallas_call(
      _stage_mid,
      grid=(GRID,),
      in_specs=[_row(D), _row(D), _row(D), _row(1), _full(1, D), _full(D, dout)],
      out_specs=_row(dout),
      out_shape=jax.ShapeDtypeStruct((NPAD, dout), jnp.float32),
  )(s0, s1, g, dis, b, w)


def _tc_out(s0, s1, g, dis, b):
  return pl.pallas_call(
      _stage_out,
      grid=(GRID,),
      in_specs=[_row(DOUT), _row(DOUT), _row(DOUT), _row(1), _full(1, DOUT)],
      out_specs=_row(DOUT),
      out_shape=jax.ShapeDtypeStruct((NPAD, DOUT), jnp.float32),
  )(s0, s1, g, dis, b)


_scatter_d = _make_edge_scatter(D)


def kernel(x, edge_index, W0, b0, W1, b1, W2, b2, W3, b3):
  src = edge_index[0].reshape(NW, NB, B)
  dst = edge_index[1].reshape(NW, NB, B)

  x_pad = jnp.pad(x, ((0, NPAD - N), (0, 0)))
  zeros_d = jnp.zeros((RPS, D), jnp.float32)
  zeros_degw = jnp.zeros((RPS, DEGW), jnp.float32)
  ones_deg = jnp.ones((B, DEGW), jnp.float32)

  deg = _degree_kernel(ones_deg, dst, zeros_degw)
  p0 = deg[0, :, 0:1]
  p1 = deg[1, :, 0:1]

  g0, dis = _tc_a(x_pad, p0, p1, W0)

  s = _scatter_d(g0, src, dst, zeros_d)
  g1 = _tc_mid(s[0], s[1], g0, dis, b0.reshape(1, D), W1, D)

  s = _scatter_d(g1, src, dst, zeros_d)
  g2 = _tc_mid(s[0], s[1], g1, dis, b1.reshape(1, D), W2, D)

  s = _scatter_d(g2, src, dst, zeros_d)
  g3 = _tc_mid(s[0], s[1], g2, dis, b2.reshape(1, D), W3, DOUT)

  g3p = jnp.pad(g3, ((0, 0), (0, D - DOUT)))
  s = _scatter_d(g3p, src, dst, zeros_d)
  out = _tc_out(s[0, :, :DOUT], s[1, :, :DOUT], g3, dis, b3.reshape(1, DOUT))

  return out[:N]


# speedup vs baseline: 3.4644x; 1.4354x over previous
import functools

import jax
import jax.numpy as jnp
from jax import lax
from jax.experimental import pallas as pl
from jax.experimental.pallas import tpu as pltpu
from jax.experimental.pallas import tpu_sc as plsc

N = 10000
NPAD = 10240
E = 320000
D = 128
HW = 64
DOUT = 64

NC = 2
NS = 16
NW = NC * NS
EPT = E // NW
B = 80
NB = EPT // B
CH = 5
NCH = NB // CH
RPS = NPAD // NS
DEGW = 128

_MESH = plsc.VectorSubcoreMesh(core_axis_name="c", subcore_axis_name="s",
                               num_cores=NC, num_subcores=NS)


def _make_edge_scatter(nh):
  @functools.partial(
      pl.kernel,
      out_type=jax.ShapeDtypeStruct((NC, nh, NPAD, HW), jnp.float32),
      mesh=_MESH,
      scratch_types=[
          pltpu.VMEM((NB, B), jnp.int32),
          pltpu.VMEM((NB, B), jnp.int32),
          [pltpu.VMEM((CH * B, HW), jnp.float32)] * 2,
          pltpu.VMEM_SHARED((NPAD, HW), jnp.float32),
          [pltpu.SemaphoreType.DMA] * 2,
      ],
      compiler_params=pltpu.CompilerParams(use_tc_tiling_on_sc=False),
  )
  def scat(*args):
    gs = args[:nh]
    (src_hbm, dst_hbm, zeros_hbm, out_hbm,
     src_v, dst_v, rows, acc, sems) = args[nh:]
    cid = lax.axis_index("c")
    sid = lax.axis_index("s")
    wid = cid * NS + sid
    pltpu.sync_copy(src_hbm.at[wid], src_v)
    pltpu.sync_copy(dst_hbm.at[wid], dst_v)

    for h in range(nh):
      g = gs[h]

      def fire(ch, buf, sem):
        for par in range(CH):
          pltpu.async_copy(g.at[src_v.at[ch * CH + par]],
                           buf.at[pl.ds(par * B, B)], sem)

      def drain_scatter(ch, buf, sem):
        for par in range(CH):
          pltpu.make_async_copy(g.at[src_v.at[ch * CH + par]],
                                buf.at[pl.ds(par * B, B)], sem).wait()
        for par in range(CH):
          pltpu.sync_copy(buf.at[pl.ds(par * B, B)],
                          acc.at[dst_v.at[ch * CH + par]], add=True)

      pltpu.sync_copy(zeros_hbm, acc.at[pl.ds(sid * RPS, RPS)])
      plsc.subcore_barrier()

      fire(0, rows[0], sems[0])

      def body(jj, carry):
        c0 = 2 * jj
        fire(c0 + 1, rows[1], sems[1])
        drain_scatter(c0, rows[0], sems[0])
        fire(c0 + 2, rows[0], sems[0])
        drain_scatter(c0 + 1, rows[1], sems[1])
        return carry

      lax.fori_loop(0, (NCH - 1) // 2, body, 0)
      drain_scatter(NCH - 1, rows[0], sems[0])

      plsc.subcore_barrier()
      pltpu.sync_copy(acc.at[pl.ds(sid * RPS, RPS)],
                      out_hbm.at[cid, h, pl.ds(sid * RPS, RPS)])
      plsc.subcore_barrier()

  return scat


@functools.partial(
    pl.kernel,
    out_type=jax.ShapeDtypeStruct((NC, NPAD, DEGW), jnp.float32),
    mesh=_MESH,
    scratch_types=[
        pltpu.VMEM((NB, B), jnp.int32),
        pltpu.VMEM((B, DEGW), jnp.float32),
        pltpu.VMEM_SHARED((NPAD, DEGW), jnp.float32),
    ],
)
def _degree_kernel(ones_hbm, dst_hbm, zeros_hbm, out_hbm,
                   dst_v, ones_v, acc):
  cid = lax.axis_index("c")
  sid = lax.axis_index("s")
  wid = cid * NS + sid
  pltpu.sync_copy(zeros_hbm, acc.at[pl.ds(sid * RPS, RPS)])
  pltpu.sync_copy(dst_hbm.at[wid], dst_v)
  pltpu.sync_copy(ones_hbm, ones_v)
  plsc.subcore_barrier()

  def body(j, carry):
    pltpu.sync_copy(ones_v, acc.at[dst_v.at[j]], add=True)
    return carry

  lax.fori_loop(0, NB, body, 0)
  plsc.subcore_barrier()
  pltpu.sync_copy(acc.at[pl.ds(sid * RPS, RPS)],
                  out_hbm.at[cid, pl.ds(sid * RPS, RPS)])


GRID = 8
BR = NPAD // GRID

_row = lambda w: pl.BlockSpec((BR, w), lambda i: (i, 0))
_full = lambda r, w: pl.BlockSpec((r, w), lambda i: (0, 0))


def _stage_a(x_ref, p0_ref, p1_ref, w_ref, glo_ref, ghi_ref, dis_ref):
  dis = lax.rsqrt(1.0 + p0_ref[...] + p1_ref[...])
  g = jnp.dot(x_ref[...], w_ref[...],
              preferred_element_type=jnp.float32) * dis
  glo_ref[...] = g[:, :HW]
  ghi_ref[...] = g[:, HW:]
  dis_ref[...] = dis


def _make_stage_mid(dout):
  def stage(s00_ref, s10_ref, s01_ref, s11_ref, glo_ref, ghi_ref,
            dis_ref, b_ref, w_ref, *out_refs):
    dis = dis_ref[...]
    b = b_ref[...]
    hlo = jnp.maximum(dis * (s00_ref[...] + s10_ref[...] + glo_ref[...])
                      + b[:, :HW], 0.0)
    hhi = jnp.maximum(dis * (s01_ref[...] + s11_ref[...] + ghi_ref[...])
                      + b[:, HW:], 0.0)
    w = w_ref[...]
    g = (jnp.dot(hlo, w[:HW, :], preferred_element_type=jnp.float32)
         + jnp.dot(hhi, w[HW:, :], preferred_element_type=jnp.float32)) * dis
    if dout == D:
      out_refs[0][...] = g[:, :HW]
      out_refs[1][...] = g[:, HW:]
    else:
      out_refs[0][...] = g
  return stage


def _stage_out(s0_ref, s1_ref, g_ref, dis_ref, b_ref, out_ref):
  o = dis_ref[...] * (s0_ref[...] + s1_ref[...] + g_ref[...]) + b_ref[...]
  m = jnp.max(o, axis=1, keepdims=True)
  e = o - m
  out_ref[...] = e - jnp.log(jnp.sum(jnp.exp(e), axis=1, keepdims=True))


def _tc_a(x, p0, p1, w):
  return pl.pallas_call(
      _stage_a,
      grid=(GRID,),
      in_specs=[_row(D), _row(1), _row(1), _full(D, D)],
      out_specs=[_row(HW), _row(HW), _row(1)],
      out_shape=[jax.ShapeDtypeStruct((NPAD, HW), jnp.float32),
                 jax.ShapeDtypeStruct((NPAD, HW), jnp.float32),
                 jax.ShapeDtypeStruct((NPAD, 1), jnp.float32)],
  )(x, p0, p1, w)


def _tc_mid(s, glo, ghi, dis, b, w, dout):
  n_out = 2 if dout == D else 1
  out = pl.pallas_call(
      _make_stage_mid(dout),
      grid=(GRID,),
      in_specs=[_row(HW)] * 6 + [_row(1), _full(1, D), _full(D, dout)],
      out_specs=[_row(HW)] * n_out,
      out_shape=[jax.ShapeDtypeStruct((NPAD, HW), jnp.float32)] * n_out,
  )(s[0, 0], s[1, 0], s[0, 1], s[1, 1], glo, ghi, dis,
    b.reshape(1, D), w)
  return out if n_out == 2 else out[0]


def _tc_out(s0, s1, g, dis, b):
  return pl.pallas_call(
      _stage_out,
      grid=(GRID,),
      in_specs=[_row(DOUT), _row(DOUT), _row(DOUT), _row(1), _full(1, DOUT)],
      out_specs=_row(DOUT),
      out_shape=jax.ShapeDtypeStruct((NPAD, DOUT), jnp.float32),
  )(s0, s1, g, dis, b)


_scatter2 = _make_edge_scatter(2)
_scatter1 = _make_edge_scatter(1)


def kernel(x, edge_index, W0, b0, W1, b1, W2, b2, W3, b3):
  src = edge_index[0].reshape(NW, NB, B)
  dst = edge_index[1].reshape(NW, NB, B)

  x_pad = jnp.pad(x, ((0, NPAD - N), (0, 0)))
  zeros_h = jnp.zeros((RPS, HW), jnp.float32)
  zeros_degw = jnp.zeros((RPS, DEGW), jnp.float32)
  ones_deg = jnp.ones((B, DEGW), jnp.float32)

  deg = _degree_kernel(ones_deg, dst, zeros_degw)
  p0 = deg[0, :, 0:1]
  p1 = deg[1, :, 0:1]

  glo, ghi, dis = _tc_a(x_pad, p0, p1, W0)

  s = _scatter2(glo, ghi, src, dst, zeros_h)
  glo, ghi = _tc_mid(s, glo, ghi, dis, b0, W1, D)

  s = _scatter2(glo, ghi, src, dst, zeros_h)
  glo, ghi = _tc_mid(s, glo, ghi, dis, b1, W2, D)

  s = _scatter2(glo, ghi, src, dst, zeros_h)
  g3 = _tc_mid(s, glo, ghi, dis, b2, W3, DOUT)

  s = _scatter1(g3, src, dst, zeros_h)
  out = _tc_out(s[0, 0], s[1, 0], g3, dis, b3.reshape(1, DOUT))

  return out[:N]


# speedup vs baseline: 3.7183x; 1.0733x over previous
import functools

import jax
import jax.numpy as jnp
from jax import lax
from jax.experimental import pallas as pl
from jax.experimental.pallas import tpu as pltpu
from jax.experimental.pallas import tpu_sc as plsc

N = 10000
NPAD = 10240
E = 320000
D = 128
HW = 64
DOUT = 64

NC = 2
NS = 16
NW = NC * NS
EPT = E // NW
B = 80
NB = EPT // B
CH = 5
NCH = NB // CH
RPS = NPAD // NS
DEGW = 8

_MESH = plsc.VectorSubcoreMesh(core_axis_name="c", subcore_axis_name="s",
                               num_cores=NC, num_subcores=NS)


def _make_edge_scatter(nh):
  @functools.partial(
      pl.kernel,
      out_type=jax.ShapeDtypeStruct((NC, nh, NPAD, HW), jnp.float32),
      mesh=_MESH,
      scratch_types=[
          pltpu.VMEM((NB, B), jnp.int32),
          pltpu.VMEM((NB, B), jnp.int32),
          [pltpu.VMEM((CH * B, HW), jnp.float32)] * 2,
          pltpu.VMEM_SHARED((NPAD, HW), jnp.float32),
          [pltpu.SemaphoreType.DMA] * 2,
      ],
      compiler_params=pltpu.CompilerParams(use_tc_tiling_on_sc=False),
  )
  def scat(*args):
    gs = args[:nh]
    (src_hbm, dst_hbm, zeros_hbm, out_hbm,
     src_v, dst_v, rows, acc, sems) = args[nh:]
    cid = lax.axis_index("c")
    sid = lax.axis_index("s")
    wid = cid * NS + sid
    pltpu.sync_copy(src_hbm.at[wid], src_v)
    pltpu.sync_copy(dst_hbm.at[wid], dst_v)

    for h in range(nh):
      g = gs[h]

      def fire(ch, buf, sem):
        for par in range(CH):
          pltpu.async_copy(g.at[src_v.at[ch * CH + par]],
                           buf.at[pl.ds(par * B, B)], sem)

      def drain_scatter(ch, buf, sem):
        for par in range(CH):
          pltpu.make_async_copy(g.at[src_v.at[ch * CH + par]],
                                buf.at[pl.ds(par * B, B)], sem).wait()
        for par in range(CH):
          pltpu.sync_copy(buf.at[pl.ds(par * B, B)],
                          acc.at[dst_v.at[ch * CH + par]], add=True)

      pltpu.sync_copy(zeros_hbm, acc.at[pl.ds(sid * RPS, RPS)])
      plsc.subcore_barrier()

      fire(0, rows[0], sems[0])

      def body(jj, carry):
        c0 = 2 * jj
        fire(c0 + 1, rows[1], sems[1])
        drain_scatter(c0, rows[0], sems[0])
        fire(c0 + 2, rows[0], sems[0])
        drain_scatter(c0 + 1, rows[1], sems[1])
        return carry

      lax.fori_loop(0, (NCH - 1) // 2, body, 0)
      drain_scatter(NCH - 1, rows[0], sems[0])

      plsc.subcore_barrier()
      pltpu.sync_copy(acc.at[pl.ds(sid * RPS, RPS)],
                      out_hbm.at[cid, h, pl.ds(sid * RPS, RPS)])
      plsc.subcore_barrier()

  return scat


@functools.partial(
    pl.kernel,
    out_type=jax.ShapeDtypeStruct((NC, NPAD, DEGW), jnp.float32),
    mesh=_MESH,
    scratch_types=[
        pltpu.VMEM((NB, B), jnp.int32),
        pltpu.VMEM((B, DEGW), jnp.float32),
        pltpu.VMEM_SHARED((NPAD, DEGW), jnp.float32),
    ],
    compiler_params=pltpu.CompilerParams(use_tc_tiling_on_sc=False),
)
def _degree_kernel(ones_hbm, dst_hbm, zeros_hbm, out_hbm,
                   dst_v, ones_v, acc):
  cid = lax.axis_index("c")
  sid = lax.axis_index("s")
  wid = cid * NS + sid
  pltpu.sync_copy(zeros_hbm, acc.at[pl.ds(sid * RPS, RPS)])
  pltpu.sync_copy(dst_hbm.at[wid], dst_v)
  pltpu.sync_copy(ones_hbm, ones_v)
  plsc.subcore_barrier()

  def body(j, carry):
    pltpu.sync_copy(ones_v, acc.at[dst_v.at[j]], add=True)
    return carry

  lax.fori_loop(0, NB, body, 0)
  plsc.subcore_barrier()
  pltpu.sync_copy(acc.at[pl.ds(sid * RPS, RPS)],
                  out_hbm.at[cid, pl.ds(sid * RPS, RPS)])


GRID = 8
BR = NPAD // GRID

_row = lambda w: pl.BlockSpec((BR, w), lambda i: (i, 0))
_full = lambda r, w: pl.BlockSpec((r, w), lambda i: (0, 0))


def _stage_a(x_ref, p0_ref, p1_ref, w_ref, glo_ref, ghi_ref, dis_ref):
  dis = lax.rsqrt(1.0 + p0_ref[...] + p1_ref[...])
  g = jnp.dot(x_ref[...], w_ref[...],
              preferred_element_type=jnp.float32) * dis
  glo_ref[...] = g[:, :HW]
  ghi_ref[...] = g[:, HW:]
  dis_ref[...] = dis


def _make_stage_mid(dout):
  def stage(s00_ref, s10_ref, s01_ref, s11_ref, glo_ref, ghi_ref,
            dis_ref, b_ref, w_ref, *out_refs):
    dis = dis_ref[...]
    b = b_ref[...]
    hlo = jnp.maximum(dis * (s00_ref[...] + s10_ref[...] + glo_ref[...])
                      + b[:, :HW], 0.0)
    hhi = jnp.maximum(dis * (s01_ref[...] + s11_ref[...] + ghi_ref[...])
                      + b[:, HW:], 0.0)
    w = w_ref[...]
    g = (jnp.dot(hlo, w[:HW, :], preferred_element_type=jnp.float32)
         + jnp.dot(hhi, w[HW:, :], preferred_element_type=jnp.float32)) * dis
    if dout == D:
      out_refs[0][...] = g[:, :HW]
      out_refs[1][...] = g[:, HW:]
    else:
      out_refs[0][...] = g
  return stage


def _stage_out(s0_ref, s1_ref, g_ref, dis_ref, b_ref, out_ref):
  o = dis_ref[...] * (s0_ref[...] + s1_ref[...] + g_ref[...]) + b_ref[...]
  m = jnp.max(o, axis=1, keepdims=True)
  e = o - m
  out_ref[...] = e - jnp.log(jnp.sum(jnp.exp(e), axis=1, keepdims=True))


def _tc_a(x, p0, p1, w):
  return pl.pallas_call(
      _stage_a,
      grid=(GRID,),
      in_specs=[_row(D), _row(1), _row(1), _full(D, D)],
      out_specs=[_row(HW), _row(HW), _row(1)],
      out_shape=[jax.ShapeDtypeStruct((NPAD, HW), jnp.float32),
                 jax.ShapeDtypeStruct((NPAD, HW), jnp.float32),
                 jax.ShapeDtypeStruct((NPAD, 1), jnp.float32)],
  )(x, p0, p1, w)


def _tc_mid(s, glo, ghi, dis, b, w, dout):
  n_out = 2 if dout == D else 1
  out = pl.pallas_call(
      _make_stage_mid(dout),
      grid=(GRID,),
      in_specs=[_row(HW)] * 6 + [_row(1), _full(1, D), _full(D, dout)],
      out_specs=[_row(HW)] * n_out,
      out_shape=[jax.ShapeDtypeStruct((NPAD, HW), jnp.float32)] * n_out,
  )(s[0, 0], s[1, 0], s[0, 1], s[1, 1], glo, ghi, dis,
    b.reshape(1, D), w)
  return out if n_out == 2 else out[0]


def _tc_out(s0, s1, g, dis, b):
  return pl.pallas_call(
      _stage_out,
      grid=(GRID,),
      in_specs=[_row(DOUT), _row(DOUT), _row(DOUT), _row(1), _full(1, DOUT)],
      out_specs=_row(DOUT),
      out_shape=jax.ShapeDtypeStruct((NPAD, DOUT), jnp.float32),
  )(s0, s1, g, dis, b)


_scatter2 = _make_edge_scatter(2)
_scatter1 = _make_edge_scatter(1)


def kernel(x, edge_index, W0, b0, W1, b1, W2, b2, W3, b3):
  src = edge_index[0].reshape(NW, NB, B)
  dst = edge_index[1].reshape(NW, NB, B)

  x_pad = jnp.pad(x, ((0, NPAD - N), (0, 0)))
  zeros_h = jnp.zeros((RPS, HW), jnp.float32)
  zeros_degw = jnp.zeros((RPS, DEGW), jnp.float32)
  ones_deg = jnp.ones((B, DEGW), jnp.float32)

  deg = _degree_kernel(ones_deg, dst, zeros_degw)
  p0 = deg[0, :, 0:1]
  p1 = deg[1, :, 0:1]

  glo, ghi, dis = _tc_a(x_pad, p0, p1, W0)

  s = _scatter2(glo, ghi, src, dst, zeros_h)
  glo, ghi = _tc_mid(s, glo, ghi, dis, b0, W1, D)

  s = _scatter2(glo, ghi, src, dst, zeros_h)
  glo, ghi = _tc_mid(s, glo, ghi, dis, b1, W2, D)

  s = _scatter2(glo, ghi, src, dst, zeros_h)
  g3 = _tc_mid(s, glo, ghi, dis, b2, W3, DOUT)

  s = _scatter1(g3, src, dst, zeros_h)
  out = _tc_out(s[0, 0], s[1, 0], g3, dis, b3.reshape(1, DOUT))

  return out[:N]


# speedup vs baseline: 4.2333x; 1.1385x over previous
import functools

import jax
import jax.numpy as jnp
from jax import lax
from jax.experimental import pallas as pl
from jax.experimental.pallas import tpu as pltpu
from jax.experimental.pallas import tpu_sc as plsc

N = 10000
NPAD = 10240
E = 320000
D = 128
HW = 64
DOUT = 64

NC = 2
NS = 16
NW = NC * NS
EPT = E // NW
B = 80
NB = EPT // B
CH = 5
NCH = NB // CH
RPS = NPAD // NS
DEGW = 8

_MESH = plsc.VectorSubcoreMesh(core_axis_name="c", subcore_axis_name="s",
                               num_cores=NC, num_subcores=NS)


def _make_edge_scatter(nh):
  @functools.partial(
      pl.kernel,
      out_type=jax.ShapeDtypeStruct((NC, NPAD, D), jnp.float32),
      mesh=_MESH,
      scratch_types=[
          pltpu.VMEM((NB, B), jnp.int32),
          pltpu.VMEM((NB, B), jnp.int32),
          [pltpu.VMEM((CH * B, HW), jnp.float32)] * 2,
          pltpu.VMEM_SHARED((NPAD, HW), jnp.float32),
          [pltpu.SemaphoreType.DMA] * 2,
      ],
      compiler_params=pltpu.CompilerParams(use_tc_tiling_on_sc=False),
  )
  def scat(*args):
    gs = args[:nh]
    (src_hbm, dst_hbm, zeros_hbm, out_hbm,
     src_v, dst_v, rows, acc, sems) = args[nh:]
    cid = lax.axis_index("c")
    sid = lax.axis_index("s")
    wid = cid * NS + sid
    pltpu.sync_copy(src_hbm.at[wid], src_v)
    pltpu.sync_copy(dst_hbm.at[wid], dst_v)

    for h in range(nh):
      g = gs[h]

      def fire(ch, buf, sem):
        for par in range(CH):
          pltpu.async_copy(g.at[src_v.at[ch * CH + par]],
                           buf.at[pl.ds(par * B, B)], sem)

      def drain_scatter(ch, buf, sem):
        for par in range(CH):
          pltpu.make_async_copy(g.at[src_v.at[ch * CH + par]],
                                buf.at[pl.ds(par * B, B)], sem).wait()
        for par in range(CH):
          pltpu.sync_copy(buf.at[pl.ds(par * B, B)],
                          acc.at[dst_v.at[ch * CH + par]], add=True)

      pltpu.sync_copy(zeros_hbm, acc.at[pl.ds(sid * RPS, RPS)])
      plsc.subcore_barrier()

      fire(0, rows[0], sems[0])

      def body(jj, carry):
        c0 = 2 * jj
        fire(c0 + 1, rows[1], sems[1])
        drain_scatter(c0, rows[0], sems[0])
        fire(c0 + 2, rows[0], sems[0])
        drain_scatter(c0 + 1, rows[1], sems[1])
        return carry

      lax.fori_loop(0, (NCH - 1) // 2, body, 0)
      drain_scatter(NCH - 1, rows[0], sems[0])

      plsc.subcore_barrier()
      pltpu.sync_copy(acc.at[pl.ds(sid * RPS, RPS)],
                      out_hbm.at[cid, pl.ds(sid * RPS, RPS),
                                 pl.ds(h * HW, HW)])
      plsc.subcore_barrier()

  return scat


@functools.partial(
    pl.kernel,
    out_type=jax.ShapeDtypeStruct((NC, NPAD, DEGW), jnp.float32),
    mesh=_MESH,
    scratch_types=[
        pltpu.VMEM((NB, B), jnp.int32),
        pltpu.VMEM((B, DEGW), jnp.float32),
        pltpu.VMEM_SHARED((NPAD, DEGW), jnp.float32),
    ],
    compiler_params=pltpu.CompilerParams(use_tc_tiling_on_sc=False),
)
def _degree_kernel(ones_hbm, dst_hbm, zeros_hbm, out_hbm,
                   dst_v, ones_v, acc):
  cid = lax.axis_index("c")
  sid = lax.axis_index("s")
  wid = cid * NS + sid
  pltpu.sync_copy(zeros_hbm, acc.at[pl.ds(sid * RPS, RPS)])
  pltpu.sync_copy(dst_hbm.at[wid], dst_v)
  pltpu.sync_copy(ones_hbm, ones_v)
  plsc.subcore_barrier()

  def body(j, carry):
    pltpu.sync_copy(ones_v, acc.at[dst_v.at[j]], add=True)
    return carry

  lax.fori_loop(0, NB, body, 0)
  plsc.subcore_barrier()
  pltpu.sync_copy(acc.at[pl.ds(sid * RPS, RPS)],
                  out_hbm.at[cid, pl.ds(sid * RPS, RPS)])


GRID = 8
BR = NPAD // GRID

_row = lambda w: pl.BlockSpec((BR, w), lambda i: (i, 0))
_full = lambda r, w: pl.BlockSpec((r, w), lambda i: (0, 0))


def _stage_a(x_ref, p0_ref, p1_ref, w_ref, glo_ref, ghi_ref, dis_ref):
  dis = lax.rsqrt(1.0 + p0_ref[...] + p1_ref[...])
  g = jnp.dot(x_ref[...], w_ref[...],
              preferred_element_type=jnp.float32) * dis
  glo_ref[...] = g[:, :HW]
  ghi_ref[...] = g[:, HW:]
  dis_ref[...] = dis


def _make_stage_mid(dout):
  def stage(s0_ref, s1_ref, glo_ref, ghi_ref,
            dis_ref, b_ref, w_ref, *out_refs):
    dis = dis_ref[...]
    b = b_ref[...]
    s = s0_ref[...] + s1_ref[...]
    hlo = jnp.maximum(dis * (s[:, :HW] + glo_ref[...]) + b[:, :HW], 0.0)
    hhi = jnp.maximum(dis * (s[:, HW:] + ghi_ref[...]) + b[:, HW:], 0.0)
    w = w_ref[...]
    g = (jnp.dot(hlo, w[:HW, :], preferred_element_type=jnp.float32)
         + jnp.dot(hhi, w[HW:, :], preferred_element_type=jnp.float32)) * dis
    if dout == D:
      out_refs[0][...] = g[:, :HW]
      out_refs[1][...] = g[:, HW:]
    else:
      out_refs[0][...] = g
  return stage


def _stage_out(s0_ref, s1_ref, g_ref, dis_ref, b_ref, out_ref):
  o = (dis_ref[...] * (s0_ref[...][:, :DOUT] + s1_ref[...][:, :DOUT]
                       + g_ref[...]) + b_ref[...])
  m = jnp.max(o, axis=1, keepdims=True)
  e = o - m
  out_ref[...] = e - jnp.log(jnp.sum(jnp.exp(e), axis=1, keepdims=True))


def _tc_a(x, p0, p1, w):
  return pl.pallas_call(
      _stage_a,
      grid=(GRID,),
      in_specs=[_row(D), _row(1), _row(1), _full(D, D)],
      out_specs=[_row(HW), _row(HW), _row(1)],
      out_shape=[jax.ShapeDtypeStruct((NPAD, HW), jnp.float32),
                 jax.ShapeDtypeStruct((NPAD, HW), jnp.float32),
                 jax.ShapeDtypeStruct((NPAD, 1), jnp.float32)],
  )(x, p0, p1, w)


def _tc_mid(s, glo, ghi, dis, b, w, dout):
  n_out = 2 if dout == D else 1
  out = pl.pallas_call(
      _make_stage_mid(dout),
      grid=(GRID,),
      in_specs=[_row(D), _row(D), _row(HW), _row(HW),
                _row(1), _full(1, D), _full(D, dout)],
      out_specs=[_row(HW)] * n_out,
      out_shape=[jax.ShapeDtypeStruct((NPAD, HW), jnp.float32)] * n_out,
  )(s[0], s[1], glo, ghi, dis, b.reshape(1, D), w)
  return out if n_out == 2 else out[0]


def _tc_out(s0, s1, g, dis, b):
  return pl.pallas_call(
      _stage_out,
      grid=(GRID,),
      in_specs=[_row(D), _row(D), _row(DOUT), _row(1), _full(1, DOUT)],
      out_specs=_row(DOUT),
      out_shape=jax.ShapeDtypeStruct((NPAD, DOUT), jnp.float32),
  )(s0, s1, g, dis, b)


_scatter2 = _make_edge_scatter(2)
_scatter1 = _make_edge_scatter(1)


def kernel(x, edge_index, W0, b0, W1, b1, W2, b2, W3, b3):
  src = edge_index[0].reshape(NW, NB, B)
  dst = edge_index[1].reshape(NW, NB, B)

  x_pad = jnp.pad(x, ((0, NPAD - N), (0, 0)))
  zeros_h = jnp.zeros((RPS, HW), jnp.float32)
  zeros_degw = jnp.zeros((RPS, DEGW), jnp.float32)
  ones_deg = jnp.ones((B, DEGW), jnp.float32)

  deg = _degree_kernel(ones_deg, dst, zeros_degw)
  p0 = deg[0, :, 0:1]
  p1 = deg[1, :, 0:1]

  glo, ghi, dis = _tc_a(x_pad, p0, p1, W0)

  s = _scatter2(glo, ghi, src, dst, zeros_h)
  glo, ghi = _tc_mid(s, glo, ghi, dis, b0, W1, D)

  s = _scatter2(glo, ghi, src, dst, zeros_h)
  glo, ghi = _tc_mid(s, glo, ghi, dis, b1, W2, D)

  s = _scatter2(glo, ghi, src, dst, zeros_h)
  g3 = _tc_mid(s, glo, ghi, dis, b2, W3, DOUT)

  s = _scatter1(g3, src, dst, zeros_h)
  out = _tc_out(s[0], s[1], g3, dis, b3.reshape(1, DOUT))

  return out[:N]


# speedup vs baseline: 4.2604x; 1.0064x over previous
import functools

import jax
import jax.numpy as jnp
from jax import lax
from jax.experimental import pallas as pl
from jax.experimental.pallas import tpu as pltpu
from jax.experimental.pallas import tpu_sc as plsc

N = 10000
NPAD = 10240
E = 320000
D = 128
HW = 64
DOUT = 64

NC = 2
NS = 16
NW = NC * NS
EPT = E // NW
B = 80
NB = EPT // B
CH = 5
NCH = NB // CH
RPS = NPAD // NS
DEGW = 8

_MESH = plsc.VectorSubcoreMesh(core_axis_name="c", subcore_axis_name="s",
                               num_cores=NC, num_subcores=NS)


def _make_edge_scatter(nh):
  @functools.partial(
      pl.kernel,
      out_type=jax.ShapeDtypeStruct((NC, NPAD, D), jnp.float32),
      mesh=_MESH,
      scratch_types=[
          pltpu.VMEM((NB, B), jnp.int32),
          pltpu.VMEM((NB, B), jnp.int32),
          [pltpu.VMEM((CH * B, HW), jnp.float32)] * 2,
          pltpu.VMEM_SHARED((NPAD, HW), jnp.float32),
          [pltpu.SemaphoreType.DMA] * 2,
      ],
      compiler_params=pltpu.CompilerParams(use_tc_tiling_on_sc=False),
  )
  def scat(*args):
    gs = args[:nh]
    (src_hbm, dst_hbm, zeros_hbm, out_hbm,
     src_v, dst_v, rows, acc, sems) = args[nh:]
    cid = lax.axis_index("c")
    sid = lax.axis_index("s")
    wid = cid * NS + sid
    pltpu.sync_copy(src_hbm.at[wid], src_v)
    pltpu.sync_copy(dst_hbm.at[wid], dst_v)

    for h in range(nh):
      g = gs[h]

      def fire(ch, buf, sem):
        for par in range(CH):
          pltpu.async_copy(g.at[src_v.at[ch * CH + par]],
                           buf.at[pl.ds(par * B, B)], sem)

      def drain_scatter(ch, buf, sem):
        for par in range(CH):
          pltpu.make_async_copy(g.at[src_v.at[ch * CH + par]],
                                buf.at[pl.ds(par * B, B)], sem).wait()
        for par in range(CH):
          pltpu.sync_copy(buf.at[pl.ds(par * B, B)],
                          acc.at[dst_v.at[ch * CH + par]], add=True)

      pltpu.sync_copy(zeros_hbm, acc.at[pl.ds(sid * RPS, RPS)])
      plsc.subcore_barrier()

      fire(0, rows[0], sems[0])

      def body(jj, carry):
        c0 = 2 * jj
        fire(c0 + 1, rows[1], sems[1])
        drain_scatter(c0, rows[0], sems[0])
        fire(c0 + 2, rows[0], sems[0])
        drain_scatter(c0 + 1, rows[1], sems[1])
        return carry

      lax.fori_loop(0, (NCH - 1) // 2, body, 0)
      drain_scatter(NCH - 1, rows[0], sems[0])

      plsc.subcore_barrier()
      pltpu.sync_copy(acc.at[pl.ds(sid * RPS, RPS)],
                      out_hbm.at[cid, pl.ds(sid * RPS, RPS),
                                 pl.ds(h * HW, HW)])
      plsc.subcore_barrier()

  return scat


@functools.partial(
    pl.kernel,
    out_type=jax.ShapeDtypeStruct((NC, NPAD, 128), jnp.float32),
    mesh=_MESH,
    scratch_types=[
        pltpu.VMEM((NB, B), jnp.int32),
        pltpu.VMEM((B, DEGW), jnp.float32),
        pltpu.VMEM_SHARED((NPAD, DEGW), jnp.float32),
    ],
    compiler_params=pltpu.CompilerParams(use_tc_tiling_on_sc=False),
)
def _degree_kernel(ones_hbm, dst_hbm, zeros_hbm, out_hbm,
                   dst_v, ones_v, acc):
  cid = lax.axis_index("c")
  sid = lax.axis_index("s")
  wid = cid * NS + sid
  pltpu.sync_copy(zeros_hbm, acc.at[pl.ds(sid * RPS, RPS)])
  pltpu.sync_copy(dst_hbm.at[wid], dst_v)
  pltpu.sync_copy(ones_hbm, ones_v)
  plsc.subcore_barrier()

  def body(j, carry):
    pltpu.sync_copy(ones_v, acc.at[dst_v.at[j]], add=True)
    return carry

  lax.fori_loop(0, NB, body, 0)
  plsc.subcore_barrier()
  pltpu.sync_copy(acc.at[pl.ds(sid * RPS, RPS)],
                  out_hbm.at[cid, pl.ds(sid * RPS, RPS), pl.ds(0, DEGW)])


GRID = 8
BR = NPAD // GRID

_row = lambda w: pl.BlockSpec((BR, w), lambda i: (i, 0))
_full = lambda r, w: pl.BlockSpec((r, w), lambda i: (0, 0))


def _stage_a(x_ref, p0_ref, p1_ref, w_ref, glo_ref, ghi_ref, dis_ref):
  dis = lax.rsqrt(1.0 + p0_ref[...] + p1_ref[...])
  g = jnp.dot(x_ref[...], w_ref[...],
              preferred_element_type=jnp.float32) * dis
  glo_ref[...] = g[:, :HW]
  ghi_ref[...] = g[:, HW:]
  dis_ref[...] = dis


def _make_stage_mid(dout):
  def stage(s0_ref, s1_ref, glo_ref, ghi_ref,
            dis_ref, b_ref, w_ref, *out_refs):
    dis = dis_ref[...]
    b = b_ref[...]
    s = s0_ref[...] + s1_ref[...]
    hlo = jnp.maximum(dis * (s[:, :HW] + glo_ref[...]) + b[:, :HW], 0.0)
    hhi = jnp.maximum(dis * (s[:, HW:] + ghi_ref[...]) + b[:, HW:], 0.0)
    w = w_ref[...]
    g = (jnp.dot(hlo, w[:HW, :], preferred_element_type=jnp.float32)
         + jnp.dot(hhi, w[HW:, :], preferred_element_type=jnp.float32)) * dis
    if dout == D:
      out_refs[0][...] = g[:, :HW]
      out_refs[1][...] = g[:, HW:]
    else:
      out_refs[0][...] = g
  return stage


def _stage_out(s0_ref, s1_ref, g_ref, dis_ref, b_ref, out_ref):
  o = (dis_ref[...] * (s0_ref[...][:, :DOUT] + s1_ref[...][:, :DOUT]
                       + g_ref[...]) + b_ref[...])
  m = jnp.max(o, axis=1, keepdims=True)
  e = o - m
  out_ref[...] = e - jnp.log(jnp.sum(jnp.exp(e), axis=1, keepdims=True))


def _tc_a(x, p0, p1, w):
  return pl.pallas_call(
      _stage_a,
      grid=(GRID,),
      in_specs=[_row(D), _row(1), _row(1), _full(D, D)],
      out_specs=[_row(HW), _row(HW), _row(1)],
      out_shape=[jax.ShapeDtypeStruct((NPAD, HW), jnp.float32),
                 jax.ShapeDtypeStruct((NPAD, HW), jnp.float32),
                 jax.ShapeDtypeStruct((NPAD, 1), jnp.float32)],
  )(x, p0, p1, w)


def _tc_mid(s, glo, ghi, dis, b, w, dout):
  n_out = 2 if dout == D else 1
  out = pl.pallas_call(
      _make_stage_mid(dout),
      grid=(GRID,),
      in_specs=[_row(D), _row(D), _row(HW), _row(HW),
                _row(1), _full(1, D), _full(D, dout)],
      out_specs=[_row(HW)] * n_out,
      out_shape=[jax.ShapeDtypeStruct((NPAD, HW), jnp.float32)] * n_out,
  )(s[0], s[1], glo, ghi, dis, b.reshape(1, D), w)
  return out if n_out == 2 else out[0]


def _tc_out(s0, s1, g, dis, b):
  return pl.pallas_call(
      _stage_out,
      grid=(GRID,),
      in_specs=[_row(D), _row(D), _row(DOUT), _row(1), _full(1, DOUT)],
      out_specs=_row(DOUT),
      out_shape=jax.ShapeDtypeStruct((NPAD, DOUT), jnp.float32),
  )(s0, s1, g, dis, b)


_scatter2 = _make_edge_scatter(2)
_scatter1 = _make_edge_scatter(1)


def kernel(x, edge_index, W0, b0, W1, b1, W2, b2, W3, b3):
  src = edge_index[0].reshape(NW, NB, B)
  dst = edge_index[1].reshape(NW, NB, B)

  x_pad = jnp.pad(x, ((0, NPAD - N), (0, 0)))
  zeros_h = jnp.zeros((RPS, HW), jnp.float32)
  zeros_degw = jnp.zeros((RPS, DEGW), jnp.float32)
  ones_deg = jnp.ones((B, DEGW), jnp.float32)

  deg = _degree_kernel(ones_deg, dst, zeros_degw)
  p0 = deg[0, :, 0:1]
  p1 = deg[1, :, 0:1]

  glo, ghi, dis = _tc_a(x_pad, p0, p1, W0)

  s = _scatter2(glo, ghi, src, dst, zeros_h)
  glo, ghi = _tc_mid(s, glo, ghi, dis, b0, W1, D)

  s = _scatter2(glo, ghi, src, dst, zeros_h)
  glo, ghi = _tc_mid(s, glo, ghi, dis, b1, W2, D)

  s = _scatter2(glo, ghi, src, dst, zeros_h)
  g3 = _tc_mid(s, glo, ghi, dis, b2, W3, DOUT)

  s = _scatter1(g3, src, dst, zeros_h)
  out = _tc_out(s[0], s[1], g3, dis, b3.reshape(1, DOUT))

  return out[:N]
